# baseline (speedup 1.0000x reference)
import functools

import jax
import jax.numpy as jnp
from jax import lax
from jax.experimental import pallas as pl
from jax.experimental.pallas import tpu as pltpu
from jax.experimental.pallas import tpu_sc as plsc

N_NODES = 10000
N_PAD = 10240
WPS = 640
D_IN = 128
D_OUT = 16
PK = 128 // D_OUT
NR = N_NODES // PK
NRP = N_PAD // PK
E_EDGES = 320000
EB = 1000
NW = 32
RW = 10
EW = RW * EB
NB = 4


def _vec_mesh():
    return plsc.VectorSubcoreMesh(core_axis_name="c", subcore_axis_name="s")


_SC_PARAMS = pltpu.CompilerParams(use_tc_tiling_on_sc=False)


def _sc_degree(edge_index):
    @functools.partial(
        pl.kernel,
        out_type=jax.ShapeDtypeStruct((2 * N_PAD,), jnp.float32),
        mesh=_vec_mesh(),
        compiler_params=_SC_PARAMS,
        scratch_types=[
            pltpu.VMEM((RW, EB), jnp.int32),
            pltpu.VMEM((EB,), jnp.float32),
            pltpu.VMEM((WPS,), jnp.float32),
            pltpu.VMEM_SHARED((N_PAD,), jnp.float32),
        ],
    )
    def k(e_hbm, out_hbm, idx_v, ones_v, zero_v, deg_sp):
        cid = lax.axis_index("c")
        sid = lax.axis_index("s")
        w = sid * 2 + cid

        for j in range(RW):
            pltpu.sync_copy(e_hbm.at[1, pl.ds(w * EW + j * EB, EB)],
                            idx_v.at[j])

        @pl.loop(0, EB, step=16)
        def _(i):
            ones_v[pl.ds(i, 16)] = jnp.ones((16,), jnp.float32)

        @pl.loop(0, WPS, step=16)
        def _(i):
            zero_v[pl.ds(i, 16)] = jnp.zeros((16,), jnp.float32)

        pltpu.sync_copy(zero_v, deg_sp.at[pl.ds(sid * WPS, WPS)])
        plsc.subcore_barrier()

        @pl.loop(0, RW)
        def _(j):
            pltpu.sync_copy(ones_v, deg_sp.at[idx_v.at[j]], add=True)

        plsc.subcore_barrier()
        pltpu.sync_copy(deg_sp.at[pl.ds(sid * WPS, WPS)],
                        out_hbm.at[pl.ds(cid * N_PAD + sid * WPS, WPS)])

    return k(edge_index)


def _dinv_packed(d8_ref, nr):
    d = d8_ref[0, :nr, :] + d8_ref[1, :nr, :] + 1.0
    dinv = lax.rsqrt(d)
    aa = lax.broadcasted_iota(jnp.int32, (8, 128), 0)
    cc = lax.broadcasted_iota(jnp.int32, (8, 128), 1) // D_OUT
    expand = (aa == cc).astype(jnp.float32)
    return jnp.dot(dinv, expand, preferred_element_type=jnp.float32)


def _tc_matmul(x3, W2):
    def body(x_ref, w_ref, h_ref):
        h_ref[...] = jnp.dot(x_ref[...], w_ref[...],
                             preferred_element_type=jnp.float32)

    return pl.pallas_call(
        body,
        out_shape=jax.ShapeDtypeStruct((NR, PK * D_IN // 8), jnp.float32),
    )(x3, W2)


def _tc_scale(h2, deg8):
    def body(h_ref, d_ref, hs_ref):
        hs_ref[...] = h_ref[...] * _dinv_packed(d_ref, NR)

    return pl.pallas_call(
        body,
        out_shape=jax.ShapeDtypeStruct((NR, 128), jnp.float32),
    )(h2, deg8)


def _sc_aggregate(hs, edge_index):
    @functools.partial(
        pl.kernel,
        out_type=jax.ShapeDtypeStruct((2, N_PAD, D_OUT), jnp.float32),
        mesh=_vec_mesh(),
        compiler_params=_SC_PARAMS,
        scratch_types=[
            pltpu.VMEM((RW, EB), jnp.int32),
            pltpu.VMEM((RW, EB), jnp.int32),
            pltpu.VMEM((NB, EB, D_OUT), jnp.float32),
            pltpu.VMEM((WPS, D_OUT), jnp.float32),
            pltpu.VMEM_SHARED((N_PAD, D_OUT), jnp.float32),
            pltpu.SemaphoreType.DMA,
            pltpu.SemaphoreType.DMA,
            pltpu.SemaphoreType.DMA,
            pltpu.SemaphoreType.DMA,
            pltpu.SemaphoreType.DMA,
            pltpu.SemaphoreType.DMA,
            pltpu.SemaphoreType.DMA,
            pltpu.SemaphoreType.DMA,
        ],
    )
    def k(hs_hbm, e_hbm, out_hbm, sidx_v, didx_v, rows_v, zero_v, agg_sp,
          g0, g1, g2, g3, s0, s1, s2, s3):
        gsem = [g0, g1, g2, g3]
        ssem = [s0, s1, s2, s3]
        cid = lax.axis_index("c")
        sid = lax.axis_index("s")
        w = sid * 2 + cid

        for j in range(RW):
            pltpu.sync_copy(e_hbm.at[0, pl.ds(w * EW + j * EB, EB)],
                            sidx_v.at[j])
            pltpu.sync_copy(e_hbm.at[1, pl.ds(w * EW + j * EB, EB)],
                            didx_v.at[j])

        @pl.loop(0, WPS)
        def _(i):
            zero_v[i, :] = jnp.zeros((D_OUT,), jnp.float32)

        pltpu.sync_copy(zero_v, agg_sp.at[pl.ds(sid * WPS, WPS)])
        plsc.subcore_barrier()

        gd = [None] * NB
        sd = [None] * NB
        pending = set()
        for j in range(min(NB - 1, RW)):
            gd[j] = pltpu.async_copy(hs_hbm.at[sidx_v.at[j]], rows_v.at[j],
                                     gsem[j])
        for j in range(RW):
            b = j % NB
            gd[b].wait()
            sd[b] = pltpu.async_copy(rows_v.at[b], agg_sp.at[didx_v.at[j]],
                                     ssem[b], add=True)
            pending.add(b)
            nj = j + NB - 1
            if nj < RW:
                nb = nj % NB
                if sd[nb] is not None:
                    sd[nb].wait()
                    pending.discard(nb)
                gd[nb] = pltpu.async_copy(hs_hbm.at[sidx_v.at[nj]],
                                          rows_v.at[nb], gsem[nb])
        for b in sorted(pending):
            sd[b].wait()

        plsc.subcore_barrier()
        pltpu.sync_copy(agg_sp.at[pl.ds(sid * WPS, WPS)],
                        out_hbm.at[cid, pl.ds(sid * WPS, WPS)])

    return k(hs, edge_index)


def _tc_final(aggpv, hs2, deg8, b2):
    def body(agg_ref, hs_ref, d_ref, b_ref, out_ref):
        a = agg_ref[0, :NR, :] + agg_ref[1, :NR, :] + hs_ref[...]
        t = a * _dinv_packed(d_ref, NR) + b_ref[...]
        t = jnp.maximum(t, 0.0)
        e = jnp.exp(t)
        ii = lax.broadcasted_iota(jnp.int32, (128, 128), 0) // D_OUT
        jj = lax.broadcasted_iota(jnp.int32, (128, 128), 1) // D_OUT
        blk = (ii == jj).astype(jnp.float32)
        gs = jnp.dot(e, blk, preferred_element_type=jnp.float32)
        out_ref[...] = t - jnp.log(gs)

    return pl.pallas_call(
        body,
        out_shape=jax.ShapeDtypeStruct((NR, 128), jnp.float32),
    )(aggpv, hs2, deg8, b2)


def kernel(x, edge_index, W, b):
    x3 = x.reshape(NR, PK * D_IN)
    W2 = jnp.kron(jnp.eye(PK, dtype=W.dtype), W)
    b2 = jnp.tile(b, (PK,)).reshape(1, 128)

    degp = _sc_degree(edge_index)
    deg8 = degp.reshape(2, NRP, PK)
    h2 = _tc_matmul(x3, W2)
    hs2 = _tc_scale(h2, deg8)
    hs_sc = hs2.reshape(N_NODES, D_OUT)
    aggp = _sc_aggregate(hs_sc, edge_index)
    aggpv = aggp.reshape(2, NRP, 128)
    out2 = _tc_final(aggpv, hs2, deg8, b2)
    return out2.reshape(N_NODES, D_OUT)


# speedup vs baseline: 1.1904x; 1.1904x over previous
import functools

import jax
import jax.numpy as jnp
from jax import lax
from jax.experimental import pallas as pl
from jax.experimental.pallas import tpu as pltpu
from jax.experimental.pallas import tpu_sc as plsc

N_NODES = 10000
N_PAD = 10240
WPS = 640
D_IN = 128
D_OUT = 16
PK = 128 // D_OUT
NR = N_NODES // PK
NRP = N_PAD // PK
E_EDGES = 320000
EB = 1000
NW = 32
RW = 10
EW = RW * EB
NB = 4


def _vec_mesh():
    return plsc.VectorSubcoreMesh(core_axis_name="c", subcore_axis_name="s")


_SC_PARAMS = pltpu.CompilerParams(use_tc_tiling_on_sc=False)


def _sc_degree(e3):
    @functools.partial(
        pl.kernel,
        out_type=jax.ShapeDtypeStruct((2 * N_PAD,), jnp.float32),
        mesh=_vec_mesh(),
        compiler_params=_SC_PARAMS,
        scratch_types=[
            pltpu.VMEM((RW, EB), jnp.int32),
            pltpu.VMEM((EB,), jnp.float32),
            pltpu.VMEM((WPS,), jnp.float32),
            pltpu.VMEM_SHARED((N_PAD,), jnp.float32),
        ],
    )
    def k(e_hbm, out_hbm, idx_v, ones_v, zero_v, deg_sp):
        cid = lax.axis_index("c")
        sid = lax.axis_index("s")
        w = sid * 2 + cid

        pltpu.sync_copy(e_hbm.at[1, pl.ds(w * RW, RW)], idx_v)

        @pl.loop(0, EB, step=16)
        def _(i):
            ones_v[pl.ds(i, 16)] = jnp.ones((16,), jnp.float32)

        @pl.loop(0, WPS, step=16)
        def _(i):
            zero_v[pl.ds(i, 16)] = jnp.zeros((16,), jnp.float32)

        pltpu.sync_copy(zero_v, deg_sp.at[pl.ds(sid * WPS, WPS)])
        plsc.subcore_barrier()

        @pl.loop(0, RW)
        def _(j):
            pltpu.sync_copy(ones_v, deg_sp.at[idx_v.at[j]], add=True)

        plsc.subcore_barrier()
        pltpu.sync_copy(deg_sp.at[pl.ds(sid * WPS, WPS)],
                        out_hbm.at[pl.ds(cid * N_PAD + sid * WPS, WPS)])

    return k(e3)


def _dinv_packed(d8_ref, nr):
    d = d8_ref[0, :nr, :] + d8_ref[1, :nr, :] + 1.0
    dinv = lax.rsqrt(d)
    aa = lax.broadcasted_iota(jnp.int32, (8, 128), 0)
    cc = lax.broadcasted_iota(jnp.int32, (8, 128), 1) // D_OUT
    expand = (aa == cc).astype(jnp.float32)
    return jnp.dot(dinv, expand, preferred_element_type=jnp.float32)


def _tc_matmul(x3, W2):
    def body(x_ref, w_ref, h_ref):
        h_ref[...] = jnp.dot(x_ref[...], w_ref[...],
                             preferred_element_type=jnp.float32)

    return pl.pallas_call(
        body,
        out_shape=jax.ShapeDtypeStruct((NR, PK * D_IN // 8), jnp.float32),
    )(x3, W2)


def _tc_scale(h2, deg8):
    def body(h_ref, d_ref, hs_ref):
        hs_ref[...] = h_ref[...] * _dinv_packed(d_ref, NR)

    return pl.pallas_call(
        body,
        out_shape=jax.ShapeDtypeStruct((NR, 128), jnp.float32),
    )(h2, deg8)


def _sc_aggregate(hs, e3):
    @functools.partial(
        pl.kernel,
        out_type=jax.ShapeDtypeStruct((2, N_PAD, D_OUT), jnp.float32),
        mesh=_vec_mesh(),
        compiler_params=_SC_PARAMS,
        scratch_types=[
            pltpu.VMEM((RW, EB), jnp.int32),
            pltpu.VMEM((RW, EB), jnp.int32),
            pltpu.VMEM((NB, EB, D_OUT), jnp.float32),
            pltpu.VMEM((WPS, D_OUT), jnp.float32),
            pltpu.VMEM_SHARED((N_PAD, D_OUT), jnp.float32),
            pltpu.SemaphoreType.DMA,
            pltpu.SemaphoreType.DMA,
            pltpu.SemaphoreType.DMA,
            pltpu.SemaphoreType.DMA,
            pltpu.SemaphoreType.DMA,
            pltpu.SemaphoreType.DMA,
            pltpu.SemaphoreType.DMA,
            pltpu.SemaphoreType.DMA,
        ],
    )
    def k(hs_hbm, e_hbm, out_hbm, sidx_v, didx_v, rows_v, zero_v, agg_sp,
          g0, g1, g2, g3, s0, s1, s2, s3):
        gsem = [g0, g1, g2, g3]
        ssem = [s0, s1, s2, s3]
        cid = lax.axis_index("c")
        sid = lax.axis_index("s")
        w = sid * 2 + cid

        pltpu.sync_copy(e_hbm.at[0, pl.ds(w * RW, RW)], sidx_v)
        pltpu.sync_copy(e_hbm.at[1, pl.ds(w * RW, RW)], didx_v)

        @pl.loop(0, WPS)
        def _(i):
            zero_v[i, :] = jnp.zeros((D_OUT,), jnp.float32)

        pltpu.sync_copy(zero_v, agg_sp.at[pl.ds(sid * WPS, WPS)])
        plsc.subcore_barrier()

        gd = [None] * NB
        sd = [None] * NB
        pending = set()
        for j in range(min(NB - 1, RW)):
            gd[j] = pltpu.async_copy(hs_hbm.at[sidx_v.at[j]], rows_v.at[j],
                                     gsem[j])
        for j in range(RW):
            b = j % NB
            gd[b].wait()
            sd[b] = pltpu.async_copy(rows_v.at[b], agg_sp.at[didx_v.at[j]],
                                     ssem[b], add=True)
            pending.add(b)
            nj = j + NB - 1
            if nj < RW:
                nb = nj % NB
                if sd[nb] is not None:
                    sd[nb].wait()
                    pending.discard(nb)
                gd[nb] = pltpu.async_copy(hs_hbm.at[sidx_v.at[nj]],
                                          rows_v.at[nb], gsem[nb])
        for b in sorted(pending):
            sd[b].wait()

        plsc.subcore_barrier()
        pltpu.sync_copy(agg_sp.at[pl.ds(sid * WPS, WPS)],
                        out_hbm.at[cid, pl.ds(sid * WPS, WPS)])

    return k(hs, e3)


def _tc_final(aggpv, hs2, deg8, b2):
    def body(agg_ref, hs_ref, d_ref, b_ref, out_ref):
        a = agg_ref[0, :NR, :] + agg_ref[1, :NR, :] + hs_ref[...]
        t = a * _dinv_packed(d_ref, NR) + b_ref[...]
        t = jnp.maximum(t, 0.0)
        e = jnp.exp(t)
        ii = lax.broadcasted_iota(jnp.int32, (128, 128), 0) // D_OUT
        jj = lax.broadcasted_iota(jnp.int32, (128, 128), 1) // D_OUT
        blk = (ii == jj).astype(jnp.float32)
        gs = jnp.dot(e, blk, preferred_element_type=jnp.float32)
        out_ref[...] = t - jnp.log(gs)

    return pl.pallas_call(
        body,
        out_shape=jax.ShapeDtypeStruct((NR, 128), jnp.float32),
    )(aggpv, hs2, deg8, b2)


def kernel(x, edge_index, W, b):
    x3 = x.reshape(NR, PK * D_IN)
    W2 = jnp.kron(jnp.eye(PK, dtype=W.dtype), W)
    b2 = jnp.tile(b, (PK,)).reshape(1, 128)

    e3 = edge_index.reshape(2, NW * RW, EB)
    degp = _sc_degree(e3)
    deg8 = degp.reshape(2, NRP, PK)
    h2 = _tc_matmul(x3, W2)
    hs2 = _tc_scale(h2, deg8)
    hs_sc = hs2.reshape(N_NODES, D_OUT)
    aggp = _sc_aggregate(hs_sc, e3)
    aggpv = aggp.reshape(2, NRP, 128)
    out2 = _tc_final(aggpv, hs2, deg8, b2)
    return out2.reshape(N_NODES, D_OUT)
